# baseline (speedup 1.0000x reference)
import functools

import jax
import jax.numpy as jnp
from jax import lax
from jax.experimental import pallas as pl
from jax.experimental.pallas import tpu as pltpu
from jax.experimental.pallas import tpu_sc as plsc

NUM_NODES = 1000000
EMBED_DIM = 64
BATCH = 16384
HALF = NUM_NODES // 2

_info = plsc.get_sparse_core_info()
_NC, _NS = _info.num_cores, _info.num_subcores
_NW = _NC * _NS
_B_PER_W = BATCH // _NW


@functools.partial(
    pl.kernel,
    mesh=plsc.VectorSubcoreMesh(core_axis_name="c", subcore_axis_name="s"),
    out_type=jax.ShapeDtypeStruct((BATCH, EMBED_DIM), jnp.float32),
    scratch_types=[
        pltpu.VMEM((_B_PER_W,), jnp.int32),
        pltpu.VMEM((_B_PER_W, EMBED_DIM), jnp.float32),
        pltpu.SemaphoreType.DMA,
    ],
    compiler_params=pltpu.CompilerParams(use_tc_tiling_on_sc=False),
)
def _gather_half(table_hbm, idx_hbm, out_hbm, idx_v, rows_v, sem):
    wid = lax.axis_index("s") * _NC + lax.axis_index("c")
    base = wid * _B_PER_W
    pltpu.sync_copy(idx_hbm.at[pl.ds(base, _B_PER_W)], idx_v)
    pltpu.async_copy(table_hbm.at[idx_v], rows_v, sem).wait()
    pltpu.sync_copy(rows_v, out_hbm.at[pl.ds(base, _B_PER_W)])


def kernel(node_indices, embeddings):
    idx = node_indices.astype(jnp.int32)
    lo = embeddings[:HALF]
    hi = embeddings[HALF:]
    idx_lo = jnp.clip(idx, 0, HALF - 1)
    idx_hi = jnp.clip(idx - HALF, 0, HALF - 1)
    out_lo = _gather_half(lo, idx_lo)
    out_hi = _gather_half(hi, idx_hi)
    return jnp.where((idx < HALF)[:, None], out_lo, out_hi)


# speedup vs baseline: 3.0361x; 3.0361x over previous
import functools

import jax
import jax.numpy as jnp
from jax import lax
from jax.experimental import pallas as pl
from jax.experimental.pallas import tpu as pltpu
from jax.experimental.pallas import tpu_sc as plsc

NUM_NODES = 1000000
EMBED_DIM = 64
BATCH = 16384

_info = plsc.get_sparse_core_info()
_NC = _info.num_cores
_B_PER_C = BATCH // _NC
_IDX_CHUNK = 1024


@functools.partial(
    pl.kernel,
    mesh=plsc.ScalarSubcoreMesh(axis_name="c", num_cores=_NC),
    out_type=jax.ShapeDtypeStruct((BATCH, EMBED_DIM), jnp.float32),
    scratch_types=[
        pltpu.SMEM((_IDX_CHUNK,), jnp.int32),
        pltpu.VMEM_SHARED((_B_PER_C, EMBED_DIM), jnp.float32),
        pltpu.SemaphoreType.DMA,
        pltpu.SemaphoreType.DMA,
    ],
)
def _gather_kernel(table_hbm, idx_hbm, out_hbm, idx_s, rows_sh, sem_i, sem):
    cid = lax.axis_index("c")
    base = cid * _B_PER_C

    for c in range(_B_PER_C // _IDX_CHUNK):
        cbase = c * _IDX_CHUNK
        pltpu.make_async_copy(
            idx_hbm.at[pl.ds(base + cbase, _IDX_CHUNK)], idx_s, sem_i
        ).start()
        pltpu.make_async_copy(
            idx_hbm.at[pl.ds(base + cbase, _IDX_CHUNK)], idx_s, sem_i
        ).wait()

        def fire(j, carry, cbase=cbase):
            i = idx_s[j]
            pltpu.make_async_copy(
                table_hbm.at[i], rows_sh.at[cbase + j], sem
            ).start()
            return carry

        lax.fori_loop(0, _IDX_CHUNK, fire, 0)

    pltpu.make_async_copy(
        table_hbm.at[pl.ds(0, _B_PER_C)], rows_sh, sem
    ).wait()
    pltpu.sync_copy(rows_sh, out_hbm.at[pl.ds(base, _B_PER_C)])


def kernel(node_indices, embeddings):
    idx = node_indices.astype(jnp.int32)
    return _gather_kernel(embeddings, idx)


# speedup vs baseline: 3.0428x; 1.0022x over previous
import functools

import jax
import jax.numpy as jnp
from jax import lax
from jax.experimental import pallas as pl
from jax.experimental.pallas import tpu as pltpu
from jax.experimental.pallas import tpu_sc as plsc

NUM_NODES = 1000000
EMBED_DIM = 64
BATCH = 16384

_info = plsc.get_sparse_core_info()
_NC = _info.num_cores
_B_PER_C = BATCH // _NC
_IDX_CHUNK = 1024


@functools.partial(
    pl.kernel,
    mesh=plsc.ScalarSubcoreMesh(axis_name="c", num_cores=_NC),
    out_type=jax.ShapeDtypeStruct((BATCH, EMBED_DIM), jnp.float32),
    scratch_types=[
        pltpu.SMEM((_IDX_CHUNK,), jnp.int32),
        pltpu.VMEM_SHARED((_B_PER_C, EMBED_DIM), jnp.float32),
        pltpu.SemaphoreType.DMA,
        pltpu.SemaphoreType.DMA,
    ],
)
def _gather_kernel(table_hbm, idx_hbm, out_hbm, idx_s, rows_sh, sem_i, sem):
    cid = lax.axis_index("c")
    base = cid * _B_PER_C

    for c in range(_B_PER_C // _IDX_CHUNK):
        cbase = c * _IDX_CHUNK
        pltpu.make_async_copy(
            idx_hbm.at[pl.ds(base + cbase, _IDX_CHUNK)], idx_s, sem_i
        ).start()
        pltpu.make_async_copy(
            idx_hbm.at[pl.ds(base + cbase, _IDX_CHUNK)], idx_s, sem_i
        ).wait()

        def fire(g, carry, cbase=cbase):
            for t in range(16):
                i = idx_s[g * 16 + t]
                pltpu.make_async_copy(
                    table_hbm.at[i], rows_sh.at[cbase + g * 16 + t], sem
                ).start()
            return carry

        lax.fori_loop(0, _IDX_CHUNK // 16, fire, 0)

    pltpu.make_async_copy(
        table_hbm.at[pl.ds(0, _B_PER_C)], rows_sh, sem
    ).wait()
    pltpu.sync_copy(rows_sh, out_hbm.at[pl.ds(base, _B_PER_C)])


def kernel(node_indices, embeddings):
    idx = node_indices.astype(jnp.int32)
    return _gather_kernel(embeddings, idx)


# speedup vs baseline: 3.2459x; 1.0667x over previous
import functools

import jax
import jax.numpy as jnp
from jax import lax
from jax.experimental import pallas as pl
from jax.experimental.pallas import tpu as pltpu
from jax.experimental.pallas import tpu_sc as plsc
from jax._src.pallas import mpmd

NUM_NODES = 1000000
EMBED_DIM = 64
BATCH = 16384

_info = plsc.get_sparse_core_info()
_NC, _NS, _L = _info.num_cores, _info.num_subcores, _info.num_lanes
_NW = _NC * _NS

SCS_ROWS = 7680
_S_PER_C = SCS_ROWS // _NC
_IDX_CHUNK = 768
TEC_ROWS = BATCH - SCS_ROWS
_T_PER_W = TEC_ROWS // _NW


def _scs_fn(table_hbm, idx_hbm, out_hbm, idx_s, rows_sh, sem_i, sem_s,
            idx_v, rows_v, sem_t):
    del idx_v, rows_v, sem_t
    cid = lax.axis_index("c")
    base = cid * _S_PER_C

    for c in range(_S_PER_C // _IDX_CHUNK):
        cbase = c * _IDX_CHUNK
        pltpu.make_async_copy(
            idx_hbm.at[pl.ds(base + cbase, _IDX_CHUNK)], idx_s, sem_i
        ).start()
        pltpu.make_async_copy(
            idx_hbm.at[pl.ds(base + cbase, _IDX_CHUNK)], idx_s, sem_i
        ).wait()

        def fire(g, carry, cbase=cbase):
            for t in range(16):
                i = idx_s[g * 16 + t]
                pltpu.make_async_copy(
                    table_hbm.at[i], rows_sh.at[cbase + g * 16 + t], sem_s
                ).start()
            return carry

        lax.fori_loop(0, _IDX_CHUNK // 16, fire, 0)

    pltpu.make_async_copy(
        table_hbm.at[pl.ds(0, _S_PER_C)], rows_sh, sem_s
    ).wait()
    pltpu.sync_copy(rows_sh, out_hbm.at[pl.ds(base, _S_PER_C)])


def _tec_fn(table_hbm, idx_hbm, out_hbm, idx_s, rows_sh, sem_i, sem_s,
            idx_v, rows_v, sem_t):
    del idx_s, rows_sh, sem_i, sem_s
    wid = lax.axis_index("s") * _NC + lax.axis_index("c")
    base = SCS_ROWS + wid * _T_PER_W
    pltpu.sync_copy(idx_hbm.at[pl.ds(base, _T_PER_W)], idx_v)

    def fire(g, carry):
        vec = idx_v[pl.ds(g * _L, _L)]
        for t in range(_L):
            i = vec[t]
            pltpu.make_async_copy(
                table_hbm.at[i], rows_v.at[g * _L + t], sem_t
            ).start()
        return carry

    lax.fori_loop(0, _T_PER_W // _L, fire, 0)
    pltpu.make_async_copy(
        table_hbm.at[pl.ds(0, _T_PER_W)], rows_v, sem_t
    ).wait()
    pltpu.sync_copy(rows_v, out_hbm.at[pl.ds(base, _T_PER_W)])


_smesh = plsc.ScalarSubcoreMesh(axis_name="c", num_cores=_NC)
_vmesh = plsc.VectorSubcoreMesh(core_axis_name="c", subcore_axis_name="s")

_gather_kernel = mpmd.mpmd_map(
    [(_smesh, _scs_fn), (_vmesh, _tec_fn)],
    out_types=jax.ShapeDtypeStruct((BATCH, EMBED_DIM), jnp.float32),
    scratch_types=[
        (pltpu.SMEM @ _smesh)((_IDX_CHUNK,), jnp.int32),
        pltpu.VMEM_SHARED((_S_PER_C, EMBED_DIM), jnp.float32),
        pltpu.SemaphoreType.DMA @ _smesh,
        pltpu.SemaphoreType.DMA @ _smesh,
        (pltpu.VMEM @ _vmesh)((_T_PER_W,), jnp.int32),
        (pltpu.VMEM @ _vmesh)((_T_PER_W, EMBED_DIM), jnp.float32),
        pltpu.SemaphoreType.DMA @ _vmesh,
    ],
)


def kernel(node_indices, embeddings):
    idx = node_indices.astype(jnp.int32)
    return _gather_kernel(embeddings, idx)
